# baseline (speedup 1.0000x reference)
import functools

import numpy as np
import jax
import jax.numpy as jnp
from jax import lax
from jax.experimental import pallas as pl
from jax.experimental.pallas import tpu as pltpu
from jax.experimental.pallas import tpu_sc as plsc

_SHAPES = [(128, 128), (64, 64), (32, 32), (16, 16), (8, 8)]
_STRIDES = [8, 16, 32, 64, 128]
_LIMITS = [(-1.0, 64.0), (64.0, 128.0), (128.0, 256.0), (256.0, 512.0), (512.0, 999999.0)]
_BIG = 99999999.0

_B, _M, _MP = 4, 50, 64
_HW = sum(h * w for h, w in _SHAPES)
_WPB = 8
_LANES = 16

_LSIZES = [h * w for h, w in _SHAPES]
_LBASES = [sum(_LSIZES[:i]) for i in range(5)]
_SEGS = [16384 // 8, 4096 // 8, 1024 // 8, 256 // 8, 16]
_VOFFS = [0, 2048, 2560, 2688, 2720]
_PER_W = 2736
_CBNDS = [128, 160, 168, 170]

_KROW = _MP * _LANES


def _build_loc_table():
    xs, ys = [], []
    for (h, w), s in zip(_SHAPES, _STRIDES):
        ix = np.arange(h * w)
        xs.append((ix % w).astype(np.float32) * s + s // 2)
        ys.append((ix // w).astype(np.float32) * s + s // 2)
    return np.stack([np.concatenate(xs), np.concatenate(ys)])


_LOC_TABLE = _build_loc_table()


def _sc_body(loc_hbm, kidx_hbm, cnt_hbm, boxes_hbm, classes_hbm,
             cls_out, ctr_out, reg_out,
             x_v, y_v, boxes_v, classes_v, kidx_v, cnt_v,
             cls_ov, ctr_ov, reg_ov, tab_v):
    wid = lax.axis_index("s") * 2 + lax.axis_index("c")
    batch = wid // _WPB
    part = wid % _WPB
    part4 = part % 4

    for lvl in range(5):
        seg = _SEGS[lvl]
        p = part4 if lvl == 4 else part
        src = _LBASES[lvl] + p * seg
        for row, dst in ((0, x_v), (1, y_v)):
            pltpu.sync_copy(loc_hbm.at[pl.ds(row * _HW + src, seg)],
                            dst.at[pl.ds(_VOFFS[lvl], seg)])
    sub = batch * _WPB + part
    pltpu.sync_copy(kidx_hbm.at[pl.ds(sub * 5 * _MP, 5 * _MP)], kidx_v)
    pltpu.sync_copy(cnt_hbm.at[pl.ds(sub * _LANES, _LANES)], cnt_v)
    pltpu.sync_copy(boxes_hbm.at[pl.ds(batch * 4 * _MP, 4 * _MP)], boxes_v)
    pltpu.sync_copy(classes_hbm.at[pl.ds(batch * _MP, _MP)], classes_v)

    for g in range(_MP // _LANES):
        x1v = boxes_v[pl.ds(0 * _MP + g * _LANES, _LANES)]
        y1v = boxes_v[pl.ds(1 * _MP + g * _LANES, _LANES)]
        x2v = boxes_v[pl.ds(2 * _MP + g * _LANES, _LANES)]
        y2v = boxes_v[pl.ds(3 * _MP + g * _LANES, _LANES)]
        cxv = (x1v + x2v) * 0.5
        cyv = (y1v + y2v) * 0.5
        areav = (x2v - x1v) * (y2v - y1v)
        for lane in range(_LANES):
            k16 = (g * _LANES + lane) * _LANES
            for r, src in enumerate((x1v, y1v, x2v, y2v, cxv, cyv, areav)):
                tab_v[pl.ds(r * _KROW + k16, _LANES)] = jnp.broadcast_to(
                    src[lane], (_LANES,))

    cntv = cnt_v[pl.ds(0, _LANES)]
    nq_l = [cntv[i] for i in range(5)]
    iota = lax.iota(jnp.int32, _LANES)

    def chunk(c, lo, hi, rad, nq, lb):
        base = c * _LANES
        xv = x_v[pl.ds(base, _LANES)]
        yv = y_v[pl.ds(base, _LANES)]

        def quad(q, st):
            best_a, best_i = st
            evv = kidx_v[pl.ds(lb + q * 4, _LANES)]
            for t in range(4):
                k = evv[t]
                kb = jnp.broadcast_to(k, (_LANES,))
                k16 = k * _LANES
                x1 = tab_v[pl.ds(0 * _KROW + k16, _LANES)]
                y1 = tab_v[pl.ds(1 * _KROW + k16, _LANES)]
                x2 = tab_v[pl.ds(2 * _KROW + k16, _LANES)]
                y2 = tab_v[pl.ds(3 * _KROW + k16, _LANES)]
                cx = tab_v[pl.ds(4 * _KROW + k16, _LANES)]
                cy = tab_v[pl.ds(5 * _KROW + k16, _LANES)]
                area = tab_v[pl.ds(6 * _KROW + k16, _LANES)]
                l = xv - x1
                tt = yv - y1
                r = x2 - xv
                b = y2 - yv
                omin = jnp.minimum(jnp.minimum(l, tt), jnp.minimum(r, b))
                omax = jnp.maximum(jnp.maximum(l, tt), jnp.maximum(r, b))
                m_c = jnp.maximum(jnp.abs(xv - cx), jnp.abs(yv - cy)) < rad
                mask = ((omin > 0.0) & (omax > lo) & (omax <= hi) & m_c)
                upd = mask & (area < best_a)
                best_a = jnp.where(upd, area, best_a)
                best_i = jnp.where(upd, kb, best_i)
            return best_a, best_i

        best_a = jnp.full((_LANES,), _BIG, jnp.float32)
        best_i = jnp.zeros((_LANES,), jnp.int32)
        best_a, best_i = lax.fori_loop(0, nq, quad, (best_a, best_i))

        pos = best_a < _BIG
        x1g = plsc.load_gather(boxes_v, [best_i])
        y1g = plsc.load_gather(boxes_v, [best_i + _MP])
        x2g = plsc.load_gather(boxes_v, [best_i + 2 * _MP])
        y2g = plsc.load_gather(boxes_v, [best_i + 3 * _MP])
        clsg = plsc.load_gather(classes_v, [best_i])
        lg = xv - x1g
        tg = yv - y1g
        rg = x2g - xv
        bg = y2g - yv
        lrmin = jnp.minimum(lg, rg)
        lrmax = jnp.maximum(lg, rg)
        tbmin = jnp.minimum(tg, bg)
        tbmax = jnp.maximum(tg, bg)
        num = jnp.where(pos, lrmin * tbmin, 1.0)
        den = jnp.where(pos, jnp.maximum(lrmax * tbmax + 1e-10, 0.0), 1.0)
        ratio = num / den
        bits = lax.bitcast_convert_type(ratio, jnp.int32)
        sq = lax.bitcast_convert_type(
            lax.shift_right_logical(bits, 1) + 0x1FBD1DF5, jnp.float32)
        for _ in range(2):
            sq = 0.5 * (sq + ratio / sq)

        sl = pl.ds(base, _LANES)
        cls_ov[sl] = jnp.where(pos, clsg, 0)
        ctr_ov[sl] = jnp.where(pos, sq, -1.0)
        reg_ov[pl.ds(0 * _PER_W + base, _LANES)] = jnp.where(pos, lg, -1.0)
        reg_ov[pl.ds(1 * _PER_W + base, _LANES)] = jnp.where(pos, tg, -1.0)
        reg_ov[pl.ds(2 * _PER_W + base, _LANES)] = jnp.where(pos, rg, -1.0)
        reg_ov[pl.ds(3 * _PER_W + base, _LANES)] = jnp.where(pos, bg, -1.0)

    def group(g, carry):
        c0 = g * 4
        lvl = ((c0 >= _CBNDS[0]).astype(jnp.int32)
               + (c0 >= _CBNDS[1]).astype(jnp.int32))

        def sel(vals, cast):
            r = cast(vals[2])
            for i in (1, 0):
                r = jnp.where(lvl == i, cast(vals[i]), r)
            return r

        lo = sel([l[0] for l in _LIMITS], jnp.float32)
        hi = sel([l[1] for l in _LIMITS], jnp.float32)
        rad = sel([s * 1.5 for s in _STRIDES], jnp.float32)
        nq = sel(nq_l, lambda v: v)
        lb = lvl * _MP
        for j in range(4):
            chunk(c0 + j, lo, hi, rad, nq, lb)
        return carry

    lax.fori_loop(0, 42, group, 0)

    for c in (168, 169):
        chunk(c, jnp.float32(_LIMITS[3][0]), jnp.float32(_LIMITS[3][1]),
              jnp.float32(_STRIDES[3] * 1.5), nq_l[3], 3 * _MP)

    @pl.when(part < 4)
    def _():
        chunk(170, jnp.float32(_LIMITS[4][0]), jnp.float32(_LIMITS[4][1]),
              jnp.float32(_STRIDES[4] * 1.5), nq_l[4], 4 * _MP)

    for lvl in range(5):
        seg = _SEGS[lvl]
        p = part4 if lvl == 4 else part
        dst = batch * _HW + _LBASES[lvl] + p * seg
        voff = _VOFFS[lvl]

        def emit(lvl=lvl, seg=seg, dst=dst, voff=voff):
            pltpu.sync_copy(cls_ov.at[pl.ds(voff, seg)],
                            cls_out.at[pl.ds(dst, seg)])
            pltpu.sync_copy(ctr_ov.at[pl.ds(voff, seg)],
                            ctr_out.at[pl.ds(dst, seg)])
            for j in range(4):
                pltpu.sync_copy(
                    reg_ov.at[pl.ds(j * _PER_W + voff, seg)],
                    reg_out.at[pl.ds((batch * 4 + j) * _HW
                                     + dst - batch * _HW, seg)])

        if lvl == 4:
            @pl.when(part < 4)
            def _():
                emit()
        else:
            emit()


@jax.jit
def _gen_targets(gt_boxes, classes):
    loc = jnp.asarray(_LOC_TABLE).reshape(-1)
    boxes_pl = jnp.transpose(gt_boxes, (0, 2, 1))
    boxes_pl = jnp.pad(boxes_pl, ((0, 0), (0, 0), (0, _MP - _M))).reshape(-1)
    classes_p = jnp.pad(classes, ((0, 0), (0, _MP - _M))).reshape(-1)

    x1, y1, x2, y2 = (gt_boxes[..., i] for i in range(4))
    maxd = jnp.maximum(x2 - x1, y2 - y1)[:, None, None, :]
    his = jnp.array([2.0 * l[1] + 2.0 for l in _LIMITS],
                    jnp.float32)[None, None, :, None]
    los = jnp.array([2.0 * _LIMITS[i][0] - 3.0 * _STRIDES[i] - 2.0
                     for i in range(5)], jnp.float32)[None, None, :, None]
    size_act = (maxd <= his) & (maxd > los)
    cy = ((y1 + y2) * 0.5)[:, None, None, :]
    rads = jnp.array([1.5 * s for s in _STRIDES],
                     jnp.float32)[None, None, :, None]
    ywin_lo = jnp.maximum(y1[:, None, None, :], cy - rads)
    ywin_hi = jnp.minimum(y2[:, None, None, :], cy + rads)
    ymin_seg = np.zeros((1, _WPB, 5, 1), np.float32)
    ymax_seg = np.zeros((1, _WPB, 5, 1), np.float32)
    for lvl, ((h, w), s) in enumerate(zip(_SHAPES, _STRIDES)):
        rpp = _SEGS[lvl] // w
        for p in range(_WPB):
            pe = (p % 4) if lvl == 4 else p
            ymin_seg[0, p, lvl, 0] = pe * rpp * s + s // 2
            ymax_seg[0, p, lvl, 0] = (pe * rpp + rpp - 1) * s + s // 2
    act = (size_act & (ywin_lo < jnp.asarray(ymax_seg) + 2.0)
           & (ywin_hi > jnp.asarray(ymin_seg) - 2.0))
    actf = act.astype(jnp.float32)
    tril = jnp.asarray(np.tril(np.ones((_M, _M), np.float32), -1))
    pos = jnp.einsum("bpsm,mM->bpsM", actf, tril.T)
    slot = jnp.arange(_MP, dtype=jnp.float32)
    onehot = actf[..., None] * (pos[..., None] == slot)
    ks = jnp.arange(_M, dtype=jnp.float32)
    comp = jnp.einsum("bpsme,m->bpse", onehot, ks)
    n = jnp.sum(act.astype(jnp.int32), axis=3)
    kidx = jnp.where(slot[None, None, None, :] < n[..., None].astype(jnp.float32),
                     comp, float(_M)).astype(jnp.int32)
    kidx = kidx.reshape(-1)
    cnt = jnp.pad((n + 3) // 4, ((0, 0), (0, 0), (0, _LANES - 5)))
    cnt = cnt.astype(jnp.int32).reshape(-1)

    mesh = plsc.VectorSubcoreMesh(core_axis_name="c", subcore_axis_name="s")
    run = functools.partial(
        pl.kernel,
        mesh=mesh,
        compiler_params=pltpu.CompilerParams(
            needs_layout_passes=False, use_tc_tiling_on_sc=False),
        out_type=[
            jax.ShapeDtypeStruct((_B * _HW,), jnp.int32),
            jax.ShapeDtypeStruct((_B * _HW,), jnp.float32),
            jax.ShapeDtypeStruct((_B * _HW * 4,), jnp.float32),
        ],
        scratch_types=[
            pltpu.VMEM((_PER_W,), jnp.float32),
            pltpu.VMEM((_PER_W,), jnp.float32),
            pltpu.VMEM((4 * _MP,), jnp.float32),
            pltpu.VMEM((_MP,), jnp.int32),
            pltpu.VMEM((5 * _MP,), jnp.int32),
            pltpu.VMEM((_LANES,), jnp.int32),
            pltpu.VMEM((_PER_W,), jnp.int32),
            pltpu.VMEM((_PER_W,), jnp.float32),
            pltpu.VMEM((_PER_W * 4,), jnp.float32),
            pltpu.VMEM((7 * _KROW,), jnp.float32),
        ],
    )(_sc_body)
    cls_p, ctr_p, reg_p = run(loc, kidx, cnt, boxes_pl, classes_p)
    reg_t = jnp.transpose(reg_p.reshape(_B, 4, _HW), (0, 2, 1))
    return cls_p.reshape(_B, _HW, 1), ctr_p.reshape(_B, _HW, 1), reg_t


def kernel(cls_logits_0, cls_logits_1, cls_logits_2, cls_logits_3, cls_logits_4,
           ctr_logits_0, ctr_logits_1, ctr_logits_2, ctr_logits_3, ctr_logits_4,
           reg_preds_0, reg_preds_1, reg_preds_2, reg_preds_3, reg_preds_4,
           gt_boxes, classes):
    return _gen_targets(gt_boxes, classes)


# speedup vs baseline: 1.0095x; 1.0095x over previous
import functools

import numpy as np
import jax
import jax.numpy as jnp
from jax import lax
from jax.experimental import pallas as pl
from jax.experimental.pallas import tpu as pltpu
from jax.experimental.pallas import tpu_sc as plsc

_SHAPES = [(128, 128), (64, 64), (32, 32), (16, 16), (8, 8)]
_STRIDES = [8, 16, 32, 64, 128]
_LIMITS = [(-1.0, 64.0), (64.0, 128.0), (128.0, 256.0), (256.0, 512.0), (512.0, 999999.0)]
_BIG = 99999999.0

_B, _M, _MP = 4, 50, 64
_HW = sum(h * w for h, w in _SHAPES)
_WPB = 8
_LANES = 16

_LSIZES = [h * w for h, w in _SHAPES]
_LBASES = [sum(_LSIZES[:i]) for i in range(5)]
_SEGS = [16384 // 8, 4096 // 8, 1024 // 8, 256 // 8, 16]
_VOFFS = [0, 2048, 2560, 2688, 2720]
_PER_W = 2736
_CBNDS = [128, 160, 168, 170]

_KROW = _MP * _LANES


def _build_loc_table():
    xs, ys = [], []
    for (h, w), s in zip(_SHAPES, _STRIDES):
        ix = np.arange(h * w)
        xs.append((ix % w).astype(np.float32) * s + s // 2)
        ys.append((ix // w).astype(np.float32) * s + s // 2)
    return np.stack([np.concatenate(xs), np.concatenate(ys)])


_LOC_TABLE = _build_loc_table()


def _sc_body(loc_hbm, kidx_hbm, cnt_hbm, boxes_hbm, classes_hbm,
             cls_out, ctr_out, reg_out,
             x_v, y_v, boxes_v, classes_v, kidx_v, cnt_v,
             cls_ov, ctr_ov, reg_ov, tab_v):
    wid = lax.axis_index("s") * 2 + lax.axis_index("c")
    batch = wid // _WPB
    part = wid % _WPB
    part4 = part % 4

    for lvl in range(5):
        seg = _SEGS[lvl]
        p = part4 if lvl == 4 else part
        src = _LBASES[lvl] + p * seg
        for row, dst in ((0, x_v), (1, y_v)):
            pltpu.sync_copy(loc_hbm.at[pl.ds(row * _HW + src, seg)],
                            dst.at[pl.ds(_VOFFS[lvl], seg)])
    sub = batch * _WPB + part
    pltpu.sync_copy(kidx_hbm.at[pl.ds(sub * 5 * _MP, 5 * _MP)], kidx_v)
    pltpu.sync_copy(cnt_hbm.at[pl.ds(sub * _LANES, _LANES)], cnt_v)
    pltpu.sync_copy(boxes_hbm.at[pl.ds(batch * 4 * _MP, 4 * _MP)], boxes_v)
    pltpu.sync_copy(classes_hbm.at[pl.ds(batch * _MP, _MP)], classes_v)

    for g in range(_MP // _LANES):
        x1v = boxes_v[pl.ds(0 * _MP + g * _LANES, _LANES)]
        y1v = boxes_v[pl.ds(1 * _MP + g * _LANES, _LANES)]
        x2v = boxes_v[pl.ds(2 * _MP + g * _LANES, _LANES)]
        y2v = boxes_v[pl.ds(3 * _MP + g * _LANES, _LANES)]
        cxv = (x1v + x2v) * 0.5
        cyv = (y1v + y2v) * 0.5
        areav = (x2v - x1v) * (y2v - y1v)
        for lane in range(_LANES):
            k16 = (g * _LANES + lane) * _LANES
            for r, src in enumerate((x1v, y1v, x2v, y2v, cxv, cyv, areav)):
                tab_v[pl.ds(r * _KROW + k16, _LANES)] = jnp.broadcast_to(
                    src[lane], (_LANES,))

    cntv = cnt_v[pl.ds(0, _LANES)]
    nq_l = [cntv[i] for i in range(5)]
    iota = lax.iota(jnp.int32, _LANES)

    def chunk(c, lo, hi, rad, nq, lb):
        base = c * _LANES
        xv = x_v[pl.ds(base, _LANES)]
        yv = y_v[pl.ds(base, _LANES)]

        def quad(q, st):
            best_a, best_i = st
            evv = kidx_v[pl.ds(lb + q * 4, _LANES)]
            for t in range(4):
                k = evv[t]
                kb = jnp.broadcast_to(k, (_LANES,))
                k16 = k * _LANES
                x1 = tab_v[pl.ds(0 * _KROW + k16, _LANES)]
                y1 = tab_v[pl.ds(1 * _KROW + k16, _LANES)]
                x2 = tab_v[pl.ds(2 * _KROW + k16, _LANES)]
                y2 = tab_v[pl.ds(3 * _KROW + k16, _LANES)]
                cx = tab_v[pl.ds(4 * _KROW + k16, _LANES)]
                cy = tab_v[pl.ds(5 * _KROW + k16, _LANES)]
                area = tab_v[pl.ds(6 * _KROW + k16, _LANES)]
                l = xv - x1
                tt = yv - y1
                r = x2 - xv
                b = y2 - yv
                omin = jnp.minimum(jnp.minimum(l, tt), jnp.minimum(r, b))
                omax = jnp.maximum(jnp.maximum(l, tt), jnp.maximum(r, b))
                m_c = jnp.maximum(jnp.abs(xv - cx), jnp.abs(yv - cy)) < rad
                mask = ((omin > 0.0) & (omax > lo) & (omax <= hi) & m_c)
                upd = mask & (area < best_a)
                best_a = jnp.where(upd, area, best_a)
                best_i = jnp.where(upd, kb, best_i)
            return best_a, best_i

        best_a = jnp.full((_LANES,), _BIG, jnp.float32)
        best_i = jnp.zeros((_LANES,), jnp.int32)
        best_a, best_i = lax.fori_loop(0, nq, quad, (best_a, best_i))

        pos = best_a < _BIG
        x1g = plsc.load_gather(boxes_v, [best_i])
        y1g = plsc.load_gather(boxes_v, [best_i + _MP])
        x2g = plsc.load_gather(boxes_v, [best_i + 2 * _MP])
        y2g = plsc.load_gather(boxes_v, [best_i + 3 * _MP])
        clsg = plsc.load_gather(classes_v, [best_i])
        lg = xv - x1g
        tg = yv - y1g
        rg = x2g - xv
        bg = y2g - yv
        lrmin = jnp.minimum(lg, rg)
        lrmax = jnp.maximum(lg, rg)
        tbmin = jnp.minimum(tg, bg)
        tbmax = jnp.maximum(tg, bg)
        num = jnp.where(pos, lrmin * tbmin, 1.0)
        den = jnp.where(pos, jnp.maximum(lrmax * tbmax + 1e-10, 0.0), 1.0)
        ratio = num / den
        bits = lax.bitcast_convert_type(ratio, jnp.int32)
        sq = lax.bitcast_convert_type(
            lax.shift_right_logical(bits, 1) + 0x1FBD1DF5, jnp.float32)
        for _ in range(2):
            sq = 0.5 * (sq + ratio / sq)

        sl = pl.ds(base, _LANES)
        cls_ov[sl] = jnp.where(pos, clsg, 0)
        ctr_ov[sl] = jnp.where(pos, sq, -1.0)
        reg_ov[pl.ds(0 * _PER_W + base, _LANES)] = jnp.where(pos, lg, -1.0)
        reg_ov[pl.ds(1 * _PER_W + base, _LANES)] = jnp.where(pos, tg, -1.0)
        reg_ov[pl.ds(2 * _PER_W + base, _LANES)] = jnp.where(pos, rg, -1.0)
        reg_ov[pl.ds(3 * _PER_W + base, _LANES)] = jnp.where(pos, bg, -1.0)

    def pair(q, carry):
        c0 = q * 2
        lvl = ((c0 >= _CBNDS[0]).astype(jnp.int32)
               + (c0 >= _CBNDS[1]).astype(jnp.int32)
               + (c0 >= _CBNDS[2]).astype(jnp.int32))

        def sel(vals, cast):
            r = cast(vals[3])
            for i in range(2, -1, -1):
                r = jnp.where(lvl == i, cast(vals[i]), r)
            return r

        lo = sel([l[0] for l in _LIMITS], jnp.float32)
        hi = sel([l[1] for l in _LIMITS], jnp.float32)
        rad = sel([s * 1.5 for s in _STRIDES], jnp.float32)
        nq = sel(nq_l, lambda v: v)
        lb = lvl * _MP
        chunk(c0, lo, hi, rad, nq, lb)
        chunk(c0 + 1, lo, hi, rad, nq, lb)
        return carry

    lax.fori_loop(0, 85, pair, 0)

    @pl.when(part < 4)
    def _():
        chunk(170, jnp.float32(_LIMITS[4][0]), jnp.float32(_LIMITS[4][1]),
              jnp.float32(_STRIDES[4] * 1.5), nq_l[4], 4 * _MP)

    for lvl in range(5):
        seg = _SEGS[lvl]
        p = part4 if lvl == 4 else part
        dst = batch * _HW + _LBASES[lvl] + p * seg
        voff = _VOFFS[lvl]

        def emit(lvl=lvl, seg=seg, dst=dst, voff=voff):
            pltpu.sync_copy(cls_ov.at[pl.ds(voff, seg)],
                            cls_out.at[pl.ds(dst, seg)])
            pltpu.sync_copy(ctr_ov.at[pl.ds(voff, seg)],
                            ctr_out.at[pl.ds(dst, seg)])
            for j in range(4):
                pltpu.sync_copy(
                    reg_ov.at[pl.ds(j * _PER_W + voff, seg)],
                    reg_out.at[pl.ds((batch * 4 + j) * _HW
                                     + dst - batch * _HW, seg)])

        if lvl == 4:
            @pl.when(part < 4)
            def _():
                emit()
        else:
            emit()


@jax.jit
def _gen_targets(gt_boxes, classes):
    loc = jnp.asarray(_LOC_TABLE).reshape(-1)
    boxes_pl = jnp.transpose(gt_boxes, (0, 2, 1))
    boxes_pl = jnp.pad(boxes_pl, ((0, 0), (0, 0), (0, _MP - _M))).reshape(-1)
    classes_p = jnp.pad(classes, ((0, 0), (0, _MP - _M))).reshape(-1)

    x1, y1, x2, y2 = (gt_boxes[..., i] for i in range(4))
    maxd = jnp.maximum(x2 - x1, y2 - y1)[:, None, None, :]
    his = jnp.array([2.0 * l[1] + 2.0 for l in _LIMITS],
                    jnp.float32)[None, None, :, None]
    los = jnp.array([2.0 * _LIMITS[i][0] - 3.0 * _STRIDES[i] - 2.0
                     for i in range(5)], jnp.float32)[None, None, :, None]
    size_act = (maxd <= his) & (maxd > los)
    cy = ((y1 + y2) * 0.5)[:, None, None, :]
    rads = jnp.array([1.5 * s for s in _STRIDES],
                     jnp.float32)[None, None, :, None]
    ywin_lo = jnp.maximum(y1[:, None, None, :], cy - rads)
    ywin_hi = jnp.minimum(y2[:, None, None, :], cy + rads)
    ymin_seg = np.zeros((1, _WPB, 5, 1), np.float32)
    ymax_seg = np.zeros((1, _WPB, 5, 1), np.float32)
    for lvl, ((h, w), s) in enumerate(zip(_SHAPES, _STRIDES)):
        rpp = _SEGS[lvl] // w
        for p in range(_WPB):
            pe = (p % 4) if lvl == 4 else p
            ymin_seg[0, p, lvl, 0] = pe * rpp * s + s // 2
            ymax_seg[0, p, lvl, 0] = (pe * rpp + rpp - 1) * s + s // 2
    act = (size_act & (ywin_lo < jnp.asarray(ymax_seg) + 2.0)
           & (ywin_hi > jnp.asarray(ymin_seg) - 2.0))
    actf = act.astype(jnp.float32)
    tril = jnp.asarray(np.tril(np.ones((_M, _M), np.float32), -1))
    pos = jnp.einsum("bpsm,mM->bpsM", actf, tril.T)
    slot = jnp.arange(_MP, dtype=jnp.float32)
    onehot = actf[..., None] * (pos[..., None] == slot)
    ks = jnp.arange(_M, dtype=jnp.float32)
    comp = jnp.einsum("bpsme,m->bpse", onehot, ks)
    n = jnp.sum(act.astype(jnp.int32), axis=3)
    kidx = jnp.where(slot[None, None, None, :] < n[..., None].astype(jnp.float32),
                     comp, float(_M)).astype(jnp.int32)
    kidx = kidx.reshape(-1)
    cnt = jnp.pad((n + 3) // 4, ((0, 0), (0, 0), (0, _LANES - 5)))
    cnt = cnt.astype(jnp.int32).reshape(-1)

    mesh = plsc.VectorSubcoreMesh(core_axis_name="c", subcore_axis_name="s")
    run = functools.partial(
        pl.kernel,
        mesh=mesh,
        compiler_params=pltpu.CompilerParams(
            needs_layout_passes=False, use_tc_tiling_on_sc=False),
        out_type=[
            jax.ShapeDtypeStruct((_B * _HW,), jnp.int32),
            jax.ShapeDtypeStruct((_B * _HW,), jnp.float32),
            jax.ShapeDtypeStruct((_B * _HW * 4,), jnp.float32),
        ],
        scratch_types=[
            pltpu.VMEM((_PER_W,), jnp.float32),
            pltpu.VMEM((_PER_W,), jnp.float32),
            pltpu.VMEM((4 * _MP,), jnp.float32),
            pltpu.VMEM((_MP,), jnp.int32),
            pltpu.VMEM((5 * _MP,), jnp.int32),
            pltpu.VMEM((_LANES,), jnp.int32),
            pltpu.VMEM((_PER_W,), jnp.int32),
            pltpu.VMEM((_PER_W,), jnp.float32),
            pltpu.VMEM((_PER_W * 4,), jnp.float32),
            pltpu.VMEM((7 * _KROW,), jnp.float32),
        ],
    )(_sc_body)
    cls_p, ctr_p, reg_p = run(loc, kidx, cnt, boxes_pl, classes_p)
    reg_t = jnp.transpose(reg_p.reshape(_B, 4, _HW), (0, 2, 1))
    return cls_p.reshape(_B, _HW, 1), ctr_p.reshape(_B, _HW, 1), reg_t


def kernel(cls_logits_0, cls_logits_1, cls_logits_2, cls_logits_3, cls_logits_4,
           ctr_logits_0, ctr_logits_1, ctr_logits_2, ctr_logits_3, ctr_logits_4,
           reg_preds_0, reg_preds_1, reg_preds_2, reg_preds_3, reg_preds_4,
           gt_boxes, classes):
    return _gen_targets(gt_boxes, classes)
